# Initial kernel scaffold; baseline (speedup 1.0000x reference)
import jax
import jax.numpy as jnp
from jax.experimental import pallas as pl


def kernel(x, W0, W1, W2, W3):
    raise NotImplementedError("write your pallas kernel here")


# speedup vs baseline: 1.4729x; 1.4729x over previous
import functools

import jax
import jax.numpy as jnp
from jax import lax
from jax.experimental import pallas as pl
from jax.experimental.pallas import tpu as pltpu, tpu_sc as plsc

VOCAB = 1000000
EMB = 64
N_PART = 4
PART = EMB // N_PART
BATCH = 16384
HIST = 50
B = BATCH * HIST

NW = 32
B_PER_W = B // NW
CHUNK = 1600
N_CHUNKS = B_PER_W // CHUNK


def _make_kernel():
    mesh = plsc.VectorSubcoreMesh(core_axis_name="c", subcore_axis_name="s")

    @functools.partial(
        pl.kernel,
        mesh=mesh,
        out_type=jax.ShapeDtypeStruct((B, EMB), jnp.float32),
        scratch_types=[
            pltpu.VMEM((CHUNK,), jnp.int32),
            pltpu.VMEM((CHUNK, PART), jnp.float32),
            pltpu.VMEM((CHUNK, PART), jnp.float32),
            pltpu.VMEM((CHUNK, PART), jnp.float32),
            pltpu.VMEM((CHUNK, PART), jnp.float32),
            pltpu.SemaphoreType.DMA,
        ],
        compiler_params=pltpu.CompilerParams(use_tc_tiling_on_sc=False),
    )
    def emb_kernel(idx_hbm, w0, w1, w2, w3, out_hbm,
                   idx_v, r0, r1, r2, r3, sem):
        wid = lax.axis_index("s") * 2 + lax.axis_index("c")
        base = wid * B_PER_W

        def body(ci, _):
            row0 = base + ci * CHUNK
            pltpu.sync_copy(idx_hbm.at[pl.ds(row0, CHUNK)], idx_v)
            c0 = pltpu.async_copy(w0.at[idx_v], r0, sem)
            c1 = pltpu.async_copy(w1.at[idx_v], r1, sem)
            c2 = pltpu.async_copy(w2.at[idx_v], r2, sem)
            c3 = pltpu.async_copy(w3.at[idx_v], r3, sem)
            c0.wait()
            c1.wait()
            c2.wait()
            c3.wait()
            pltpu.sync_copy(r0, out_hbm.at[pl.ds(row0, CHUNK), pl.ds(0, PART)])
            pltpu.sync_copy(r1, out_hbm.at[pl.ds(row0, CHUNK), pl.ds(PART, PART)])
            pltpu.sync_copy(r2, out_hbm.at[pl.ds(row0, CHUNK), pl.ds(2 * PART, PART)])
            pltpu.sync_copy(r3, out_hbm.at[pl.ds(row0, CHUNK), pl.ds(3 * PART, PART)])
            return ()

        lax.fori_loop(0, N_CHUNKS, body, ())

    return emb_kernel


_emb_kernel = _make_kernel()


def kernel(x, W0, W1, W2, W3):
    idx = x.reshape(-1).astype(jnp.int32)
    out = _emb_kernel(idx, W0, W1, W2, W3)
    return out.reshape(BATCH, HIST, EMB)


# speedup vs baseline: 1.7892x; 1.2148x over previous
import functools

import jax
import jax.numpy as jnp
from jax import lax
from jax.experimental import pallas as pl
from jax.experimental.pallas import tpu as pltpu, tpu_sc as plsc

VOCAB = 1000000
EMB = 64
N_PART = 4
PART = EMB // N_PART
BATCH = 16384
HIST = 50
B = BATCH * HIST

NW = 32
B_PER_W = B // NW
CHUNK = 1600
N_CHUNKS = B_PER_W // CHUNK

TR_N = 8192
TR_STEPS = (VOCAB + TR_N - 1) // TR_N


def _tr_body(in_ref, out_ref):
    x = in_ref[...]
    t = jnp.transpose(x, (1, 0))
    t3 = t.reshape(TR_N // 8, 8, PART)
    for r in range(8):
        out_ref[:, PART * r : PART * (r + 1)] = t3[:, r, :]


def _relayout(wt):
    q = pl.pallas_call(
        _tr_body,
        grid=(TR_STEPS,),
        in_specs=[pl.BlockSpec((16, TR_N), lambda i: (0, i))],
        out_specs=pl.BlockSpec((TR_N // 8, 128), lambda i: (i, 0)),
        out_shape=jax.ShapeDtypeStruct((VOCAB // 8, 128), jnp.float32),
    )(wt)
    return q.reshape(VOCAB, PART)


def _make_gather():
    mesh = plsc.VectorSubcoreMesh(core_axis_name="c", subcore_axis_name="s")

    @functools.partial(
        pl.kernel,
        mesh=mesh,
        out_type=jax.ShapeDtypeStruct((B, EMB), jnp.float32),
        scratch_types=[
            pltpu.VMEM((CHUNK,), jnp.int32),
            pltpu.VMEM((CHUNK, PART), jnp.float32),
            pltpu.VMEM((CHUNK, PART), jnp.float32),
            pltpu.VMEM((CHUNK, PART), jnp.float32),
            pltpu.VMEM((CHUNK, PART), jnp.float32),
            pltpu.SemaphoreType.DMA,
        ],
        compiler_params=pltpu.CompilerParams(use_tc_tiling_on_sc=False),
    )
    def emb_kernel(idx_hbm, w0, w1, w2, w3, out_hbm,
                   idx_v, r0, r1, r2, r3, sem):
        wid = lax.axis_index("s") * 2 + lax.axis_index("c")
        base = wid * B_PER_W

        def body(ci, _):
            row0 = base + ci * CHUNK
            pltpu.sync_copy(idx_hbm.at[pl.ds(row0, CHUNK)], idx_v)
            c0 = pltpu.async_copy(w0.at[idx_v], r0, sem)
            c1 = pltpu.async_copy(w1.at[idx_v], r1, sem)
            c2 = pltpu.async_copy(w2.at[idx_v], r2, sem)
            c3 = pltpu.async_copy(w3.at[idx_v], r3, sem)
            c0.wait()
            c1.wait()
            c2.wait()
            c3.wait()
            pltpu.sync_copy(r0, out_hbm.at[pl.ds(row0, CHUNK), pl.ds(0, PART)])
            pltpu.sync_copy(r1, out_hbm.at[pl.ds(row0, CHUNK), pl.ds(PART, PART)])
            pltpu.sync_copy(r2, out_hbm.at[pl.ds(row0, CHUNK), pl.ds(2 * PART, PART)])
            pltpu.sync_copy(r3, out_hbm.at[pl.ds(row0, CHUNK), pl.ds(3 * PART, PART)])
            return ()

        lax.fori_loop(0, N_CHUNKS, body, ())

    return emb_kernel


_gather = _make_gather()


def kernel(x, W0, W1, W2, W3):
    idx = x.reshape(-1).astype(jnp.int32)
    q0 = _relayout(W0.T)
    q1 = _relayout(W1.T)
    q2 = _relayout(W2.T)
    q3 = _relayout(W3.T)
    out = _gather(idx, q0, q1, q2, q3)
    return out.reshape(BATCH, HIST, EMB)
